# Initial kernel scaffold; baseline (speedup 1.0000x reference)
import jax
import jax.numpy as jnp
from jax.experimental import pallas as pl


def kernel(code_tensor, word_edge, line_edge, emb, W1, b1, g1, be1, attW, attb, ctx, W2, b2, g2, be2, sattW, sattb, sctx, fcW, fcb):
    raise NotImplementedError("write your pallas kernel here")


# speedup vs baseline: 63.0620x; 63.0620x over previous
import functools

import jax
import jax.numpy as jnp
from jax import lax
from jax.experimental import pallas as pl
from jax.experimental.pallas import tpu as pltpu
from jax.experimental.pallas import tpu_sc as plsc

NF = 16
NL = 64
S = 50
SCG = NF * NL
NND = SCG * S
EPG = 800
H1 = 64
H2 = 64
A1 = 32
A2 = 32
GB = 8


def _embw_body(emb_ref, w1_ref, out_ref):
    out_ref[...] = jnp.dot(emb_ref[...], w1_ref[...],
                           preferred_element_type=jnp.float32)


def _tc_embw(emb, W1):
    v = emb.shape[0]
    vb = 2000
    return pl.pallas_call(
        _embw_body,
        grid=(v // vb,),
        in_specs=[
            pl.BlockSpec((vb, S), lambda i: (i, 0)),
            pl.BlockSpec((S, H1), lambda i: (0, 0)),
        ],
        out_specs=pl.BlockSpec((vb, H1), lambda i: (i, 0)),
        out_shape=jax.ShapeDtypeStruct((v, H1), jnp.float32),
    )(emb, W1)


def _sc_gather(table, tokens):
    info = plsc.get_sparse_core_info()
    nw = info.num_cores * info.num_subcores
    bpw = NND // nw

    mesh = plsc.VectorSubcoreMesh(core_axis_name="c", subcore_axis_name="s")

    @functools.partial(
        pl.kernel,
        mesh=mesh,
        out_type=jax.ShapeDtypeStruct((NND, H1), jnp.float32),
        scratch_types=[
            pltpu.VMEM((bpw,), jnp.int32),
            pltpu.VMEM((bpw, H1), jnp.float32),
            pltpu.SemaphoreType.DMA,
        ],
        compiler_params=pltpu.CompilerParams(use_tc_tiling_on_sc=False),
    )
    def k(table_hbm, idx_hbm, out_hbm, idx_v, rows_v, sem):
        wid = lax.axis_index("s") * info.num_cores + lax.axis_index("c")
        base = wid * bpw
        pltpu.sync_copy(idx_hbm.at[pl.ds(base, bpw)], idx_v)
        pltpu.async_copy(table_hbm.at[idx_v], rows_v, sem).wait()
        pltpu.sync_copy(rows_v, out_hbm.at[pl.ds(base, bpw)])

    return k(table, tokens)


def _sc_counts(edges_blk):
    info = plsc.get_sparse_core_info()
    nw = info.num_cores * info.num_subcores
    gpt = SCG // nw
    rl = 16
    rounds = gpt // rl
    ewords = 2 * EPG * rl
    cwords = rl * S * S

    mesh = plsc.VectorSubcoreMesh(core_axis_name="c", subcore_axis_name="s")

    @functools.partial(
        pl.kernel,
        mesh=mesh,
        out_type=jax.ShapeDtypeStruct((SCG * S * S,), jnp.float32),
        scratch_types=[
            pltpu.VMEM((ewords,), jnp.int32),
            pltpu.VMEM((cwords,), jnp.float32),
        ],
        compiler_params=pltpu.CompilerParams(needs_layout_passes=False),
    )
    def k(edges_hbm, out_hbm, ebuf, cbuf):
        wid = lax.axis_index("s") * info.num_cores + lax.axis_index("c")
        lane_off = lax.iota(jnp.int32, 16) * (S * S)
        zeros16 = jnp.zeros((16,), jnp.float32)
        ones16 = jnp.ones((16,), jnp.float32)
        for r in range(rounds):
            blk = wid * rounds + r
            pltpu.sync_copy(edges_hbm.at[pl.ds(blk * ewords, ewords)], ebuf)

            def zero_body(i, _):
                cbuf[pl.ds(i * 16, 16)] = zeros16
                return ()

            lax.fori_loop(0, cwords // 16, zero_body, ())

            def edge_body(e, _):
                s_v = ebuf[pl.ds(e * 16, 16)]
                d_v = ebuf[pl.ds((EPG + e) * 16, 16)]
                idx = lane_off + d_v * S + s_v
                plsc.addupdate_scatter(cbuf, [idx], ones16)
                return ()

            lax.fori_loop(0, EPG, edge_body, ())
            pltpu.sync_copy(cbuf, out_hbm.at[pl.ds(blk * cwords, cwords)])

    return k(edges_blk)


def _word_body(c_ref, xw_ref, b1_ref, g1_ref, be1_ref, attw_ref,
               attb_ref, ctx_ref, w2_ref, b2_ref,
               wout_ref, sents_ref, line_ref):
    cnt = c_ref[...]
    deg = jnp.sum(cnt, axis=2) + 1.0
    dinv = lax.rsqrt(deg)
    adj = cnt * dinv[:, :, None] * dinv[:, None, :]
    eye = (lax.broadcasted_iota(jnp.int32, (S, S), 0)
           == lax.broadcasted_iota(jnp.int32, (S, S), 1)).astype(jnp.float32)
    adj = adj + eye[None] * (dinv * dinv)[:, :, None]

    xw3 = xw_ref[...]
    wo = lax.dot_general(adj, xw3, (((2,), (1,)), ((0,), (0,))),
                         preferred_element_type=jnp.float32)
    wo = wo + b1_ref[...][None, None, :]

    mu = jnp.mean(wo, axis=2, keepdims=True)
    var = jnp.mean((wo - mu) ** 2, axis=2, keepdims=True)
    normed = (wo - mu) * lax.rsqrt(var + 1e-5) * g1_ref[...][None, None, :] \
        + be1_ref[...][None, None, :]

    t = jnp.tanh(jnp.dot(normed.reshape(GB * S, H1), attw_ref[...],
                         preferred_element_type=jnp.float32)
                 + attb_ref[...][None, :])
    t3 = t.reshape(GB, S, A1)
    att = jnp.sum(t3 * ctx_ref[...][None, None, :], axis=2, keepdims=True)
    mx = jnp.max(att, axis=1, keepdims=True)
    ex = jnp.exp(att - mx)
    w = ex / jnp.sum(ex, axis=1, keepdims=True)
    wout_ref[...] = w

    sents = wo * w
    sents_ref[...] = sents

    u = jnp.sum(adj, axis=1, keepdims=True)
    vsum = lax.dot_general(u, sents, (((2,), (1,)), ((0,), (0,))),
                           preferred_element_type=jnp.float32)
    line = jnp.dot(vsum.reshape(GB, H1), w2_ref[...],
                   preferred_element_type=jnp.float32) \
        + float(S) * b2_ref[...][None, :]
    line_ref[...] = line


def _sent_body(line_ref, g2_ref, be2_ref, sattw_ref, sattb_ref, sctx_ref,
               fcw_ref, fcb_ref, sw_ref, scores_ref):
    line = line_ref[...]
    mu = jnp.mean(line, axis=1, keepdims=True)
    var = jnp.mean((line - mu) ** 2, axis=1, keepdims=True)
    normed = (line - mu) * lax.rsqrt(var + 1e-5) * g2_ref[...][None, :] \
        + be2_ref[...][None, :]
    t = jnp.tanh(jnp.dot(normed, sattw_ref[...],
                         preferred_element_type=jnp.float32)
                 + sattb_ref[...][None, :])
    att = jnp.sum(t * sctx_ref[...][None, :], axis=1, keepdims=True)
    att3 = att.reshape(NF, NL, 1)
    mx = jnp.max(att3, axis=1, keepdims=True)
    ex = jnp.exp(att3 - mx)
    sw = ex / jnp.sum(ex, axis=1, keepdims=True)
    sw_ref[...] = sw
    code = jnp.sum(line.reshape(NF, NL, H2) * sw, axis=1)
    scores = jnp.dot(code, fcw_ref[...], preferred_element_type=jnp.float32) \
        + fcb_ref[...][None, :]
    scores_ref[...] = 1.0 / (1.0 + jnp.exp(-scores))


def kernel(code_tensor, word_edge, line_edge, emb, W1, b1, g1, be1, attW,
           attb, ctx, W2, b2, g2, be2, sattW, sattb, sctx, fcW, fcb):
    del line_edge
    tokens = code_tensor.reshape(NND).astype(jnp.int32)
    embw = _tc_embw(emb, W1)
    xw = _sc_gather(embw, tokens).reshape(SCG, S, H1)

    edges_blk = (word_edge.reshape(64, 16, 2 * EPG).astype(jnp.int32)
                 .transpose(0, 2, 1).reshape(-1))
    counts = _sc_counts(edges_blk).reshape(SCG, S, S)

    nsteps = SCG // GB
    wspec = [
        pl.BlockSpec((H1,), lambda i: (0,)),
        pl.BlockSpec((H1,), lambda i: (0,)),
        pl.BlockSpec((H1,), lambda i: (0,)),
        pl.BlockSpec((H1, A1), lambda i: (0, 0)),
        pl.BlockSpec((A1,), lambda i: (0,)),
        pl.BlockSpec((A1,), lambda i: (0,)),
        pl.BlockSpec((H1, H2), lambda i: (0, 0)),
        pl.BlockSpec((H2,), lambda i: (0,)),
    ]
    wout, sents, line_out = pl.pallas_call(
        _word_body,
        grid=(nsteps,),
        in_specs=[
            pl.BlockSpec((GB, S, S), lambda i: (i, 0, 0)),
            pl.BlockSpec((GB, S, H1), lambda i: (i, 0, 0)),
        ] + wspec,
        out_specs=[
            pl.BlockSpec((GB, S, 1), lambda i: (i, 0, 0)),
            pl.BlockSpec((GB, S, H1), lambda i: (i, 0, 0)),
            pl.BlockSpec((GB, H2), lambda i: (i, 0)),
        ],
        out_shape=[
            jax.ShapeDtypeStruct((SCG, S, 1), jnp.float32),
            jax.ShapeDtypeStruct((SCG, S, H1), jnp.float32),
            jax.ShapeDtypeStruct((SCG, H2), jnp.float32),
        ],
    )(counts, xw, b1, g1, be1, attW, attb, ctx, W2, b2)

    sw, scores = pl.pallas_call(
        _sent_body,
        out_shape=[
            jax.ShapeDtypeStruct((NF, NL, 1), jnp.float32),
            jax.ShapeDtypeStruct((NF, 1), jnp.float32),
        ],
    )(line_out, g2, be2, sattW, sattb, sctx, fcW, fcb)

    word_att = wout.reshape(NF, NL, S)
    return (scores, word_att, sw.reshape(NF, NL), sents)


# speedup vs baseline: 66.7829x; 1.0590x over previous
import functools

import jax
import jax.numpy as jnp
from jax import lax
from jax.experimental import pallas as pl
from jax.experimental.pallas import tpu as pltpu
from jax.experimental.pallas import tpu_sc as plsc

NF = 16
NL = 64
S = 50
SCG = NF * NL
NND = SCG * S
EPG = 800
H1 = 64
H2 = 64
A1 = 32
A2 = 32
GB = 8


def _embw_body(emb_ref, w1_ref, out_ref):
    out_ref[...] = jnp.dot(emb_ref[...], w1_ref[...],
                           preferred_element_type=jnp.float32)


def _tc_embw(emb, W1):
    v = emb.shape[0]
    vb = 2000
    return pl.pallas_call(
        _embw_body,
        grid=(v // vb,),
        in_specs=[
            pl.BlockSpec((vb, S), lambda i: (i, 0)),
            pl.BlockSpec((S, H1), lambda i: (0, 0)),
        ],
        out_specs=pl.BlockSpec((vb, H1), lambda i: (i, 0)),
        out_shape=jax.ShapeDtypeStruct((v, H1), jnp.float32),
    )(emb, W1)


def _sc_gather(table, tokens):
    info = plsc.get_sparse_core_info()
    nw = info.num_cores * info.num_subcores
    bpw = NND // nw

    mesh = plsc.VectorSubcoreMesh(core_axis_name="c", subcore_axis_name="s")

    @functools.partial(
        pl.kernel,
        mesh=mesh,
        out_type=jax.ShapeDtypeStruct((NND, H1), jnp.float32),
        scratch_types=[
            pltpu.VMEM((bpw,), jnp.int32),
            pltpu.VMEM((bpw, H1), jnp.float32),
            pltpu.SemaphoreType.DMA,
        ],
        compiler_params=pltpu.CompilerParams(use_tc_tiling_on_sc=False),
    )
    def k(table_hbm, idx_hbm, out_hbm, idx_v, rows_v, sem):
        wid = lax.axis_index("s") * info.num_cores + lax.axis_index("c")
        base = wid * bpw
        pltpu.sync_copy(idx_hbm.at[pl.ds(base, bpw)], idx_v)
        pltpu.async_copy(table_hbm.at[idx_v], rows_v, sem).wait()
        pltpu.sync_copy(rows_v, out_hbm.at[pl.ds(base, bpw)])

    return k(table, tokens)


def _sc_counts(edges_flat, zeros_blk):
    info = plsc.get_sparse_core_info()
    nw = info.num_cores * info.num_subcores
    gpt = SCG // nw
    rl = 16
    rounds = gpt // rl
    ewords = 2 * EPG * rl

    mesh = plsc.VectorSubcoreMesh(core_axis_name="c", subcore_axis_name="s")

    @functools.partial(
        pl.kernel,
        mesh=mesh,
        out_type=jax.ShapeDtypeStruct((SCG, S, S), jnp.float32),
        scratch_types=[
            pltpu.VMEM((ewords,), jnp.int32),
            pltpu.VMEM((rl, S, S), jnp.float32),
        ],
        compiler_params=pltpu.CompilerParams(
            needs_layout_passes=False, use_tc_tiling_on_sc=False),
    )
    def k(edges_hbm, zeros_hbm, out_hbm, ebuf, cbuf):
        wid = lax.axis_index("s") * info.num_cores + lax.axis_index("c")
        lanes = lax.iota(jnp.int32, 16)
        lane_base = lanes * (2 * EPG)
        ones16 = jnp.ones((16,), jnp.float32)
        for r in range(rounds):
            blk = wid * rounds + r
            pltpu.sync_copy(edges_hbm.at[pl.ds(blk * ewords, ewords)], ebuf)
            pltpu.sync_copy(zeros_hbm, cbuf)

            def edge_body(e, _):
                s_v = plsc.load_gather(ebuf, [lane_base + e])
                d_v = plsc.load_gather(ebuf, [lane_base + (EPG + e)])
                plsc.addupdate_scatter(cbuf, [lanes, d_v, s_v], ones16)
                return ()

            lax.fori_loop(0, EPG, edge_body, ())
            pltpu.sync_copy(cbuf, out_hbm.at[pl.ds(blk * rl, rl)])

    return k(edges_flat, zeros_blk)


def _word_body(c_ref, xw_ref, b1_ref, g1_ref, be1_ref, attw_ref,
               attb_ref, ctx_ref, w2_ref, b2_ref,
               wout_ref, sents_ref, line_ref):
    cnt = c_ref[...]
    deg = jnp.sum(cnt, axis=2) + 1.0
    dinv = lax.rsqrt(deg)
    adj = cnt * dinv[:, :, None] * dinv[:, None, :]
    eye = (lax.broadcasted_iota(jnp.int32, (S, S), 0)
           == lax.broadcasted_iota(jnp.int32, (S, S), 1)).astype(jnp.float32)
    adj = adj + eye[None] * (dinv * dinv)[:, :, None]

    xw3 = xw_ref[...].reshape(GB, S, H1)
    wo = lax.dot_general(adj, xw3, (((2,), (1,)), ((0,), (0,))),
                         preferred_element_type=jnp.float32)
    wo = wo + b1_ref[...][None, None, :]

    mu = jnp.mean(wo, axis=2, keepdims=True)
    var = jnp.mean((wo - mu) ** 2, axis=2, keepdims=True)
    normed = (wo - mu) * lax.rsqrt(var + 1e-5) * g1_ref[...][None, None, :] \
        + be1_ref[...][None, None, :]

    t = jnp.tanh(jnp.dot(normed.reshape(GB * S, H1), attw_ref[...],
                         preferred_element_type=jnp.float32)
                 + attb_ref[...][None, :])
    t3 = t.reshape(GB, S, A1)
    att = jnp.sum(t3 * ctx_ref[...][None, None, :], axis=2, keepdims=True)
    mx = jnp.max(att, axis=1, keepdims=True)
    ex = jnp.exp(att - mx)
    w = ex / jnp.sum(ex, axis=1, keepdims=True)
    wout_ref[...] = w

    sents = wo * w
    sents_ref[...] = sents

    u = jnp.sum(adj, axis=1, keepdims=True)
    vsum = lax.dot_general(u, sents, (((2,), (1,)), ((0,), (0,))),
                           preferred_element_type=jnp.float32)
    line = jnp.dot(vsum.reshape(GB, H1), w2_ref[...],
                   preferred_element_type=jnp.float32) \
        + float(S) * b2_ref[...][None, :]
    line_ref[...] = line


def _sent_body(line_ref, g2_ref, be2_ref, sattw_ref, sattb_ref, sctx_ref,
               fcw_ref, fcb_ref, sw_ref, scores_ref):
    line = line_ref[...]
    mu = jnp.mean(line, axis=1, keepdims=True)
    var = jnp.mean((line - mu) ** 2, axis=1, keepdims=True)
    normed = (line - mu) * lax.rsqrt(var + 1e-5) * g2_ref[...][None, :] \
        + be2_ref[...][None, :]
    t = jnp.tanh(jnp.dot(normed, sattw_ref[...],
                         preferred_element_type=jnp.float32)
                 + sattb_ref[...][None, :])
    att = jnp.sum(t * sctx_ref[...][None, :], axis=1, keepdims=True)
    att3 = att.reshape(NF, NL, 1)
    mx = jnp.max(att3, axis=1, keepdims=True)
    ex = jnp.exp(att3 - mx)
    sw = ex / jnp.sum(ex, axis=1, keepdims=True)
    sw_ref[...] = sw
    code = jnp.sum(line.reshape(NF, NL, H2) * sw, axis=1)
    scores = jnp.dot(code, fcw_ref[...], preferred_element_type=jnp.float32) \
        + fcb_ref[...][None, :]
    scores_ref[...] = 1.0 / (1.0 + jnp.exp(-scores))


def kernel(code_tensor, word_edge, line_edge, emb, W1, b1, g1, be1, attW,
           attb, ctx, W2, b2, g2, be2, sattW, sattb, sctx, fcW, fcb):
    del line_edge
    tokens = code_tensor.reshape(NND).astype(jnp.int32)
    embw = _tc_embw(emb, W1)
    xw = _sc_gather(embw, tokens)

    edges_flat = word_edge.reshape(-1).astype(jnp.int32)
    zeros_blk = jnp.zeros((16, S, S), jnp.float32)
    counts = _sc_counts(edges_flat, zeros_blk)

    nsteps = SCG // GB
    wspec = [
        pl.BlockSpec((H1,), lambda i: (0,)),
        pl.BlockSpec((H1,), lambda i: (0,)),
        pl.BlockSpec((H1,), lambda i: (0,)),
        pl.BlockSpec((H1, A1), lambda i: (0, 0)),
        pl.BlockSpec((A1,), lambda i: (0,)),
        pl.BlockSpec((A1,), lambda i: (0,)),
        pl.BlockSpec((H1, H2), lambda i: (0, 0)),
        pl.BlockSpec((H2,), lambda i: (0,)),
    ]
    wout, sents, line_out = pl.pallas_call(
        _word_body,
        grid=(nsteps,),
        in_specs=[
            pl.BlockSpec((GB, S, S), lambda i: (i, 0, 0)),
            pl.BlockSpec((GB * S, H1), lambda i: (i, 0)),
        ] + wspec,
        out_specs=[
            pl.BlockSpec((GB, S, 1), lambda i: (i, 0, 0)),
            pl.BlockSpec((GB, S, H1), lambda i: (i, 0, 0)),
            pl.BlockSpec((GB, H2), lambda i: (i, 0)),
        ],
        out_shape=[
            jax.ShapeDtypeStruct((SCG, S, 1), jnp.float32),
            jax.ShapeDtypeStruct((SCG, S, H1), jnp.float32),
            jax.ShapeDtypeStruct((SCG, H2), jnp.float32),
        ],
    )(counts, xw, b1, g1, be1, attW, attb, ctx, W2, b2)

    sw, scores = pl.pallas_call(
        _sent_body,
        out_shape=[
            jax.ShapeDtypeStruct((NF, NL, 1), jnp.float32),
            jax.ShapeDtypeStruct((NF, 1), jnp.float32),
        ],
    )(line_out, g2, be2, sattW, sattb, sctx, fcW, fcb)

    word_att = wout.reshape(NF, NL, S)
    return (scores, word_att, sw.reshape(NF, NL), sents)


# speedup vs baseline: 72.9780x; 1.0928x over previous
import functools

import jax
import jax.numpy as jnp
from jax import lax
from jax.experimental import pallas as pl
from jax.experimental.pallas import tpu as pltpu
from jax.experimental.pallas import tpu_sc as plsc

NF = 16
NL = 64
S = 50
SCG = NF * NL
NND = SCG * S
EPG = 800
H1 = 64
H2 = 64
A1 = 32
A2 = 32
GB = 8


def _embw_body(emb_ref, w1_ref, out_ref):
    out_ref[:, :H1] = jnp.dot(emb_ref[...], w1_ref[...],
                              preferred_element_type=jnp.float32)
    out_ref[:, H1:] = jnp.zeros_like(out_ref[:, H1:])


def _tc_embw(emb, W1):
    v = emb.shape[0]
    vb = 2000
    return pl.pallas_call(
        _embw_body,
        grid=(v // vb,),
        in_specs=[
            pl.BlockSpec((vb, S), lambda i: (i, 0)),
            pl.BlockSpec((S, H1), lambda i: (0, 0)),
        ],
        out_specs=pl.BlockSpec((vb, 128), lambda i: (i, 0)),
        out_shape=jax.ShapeDtypeStruct((v, 128), jnp.float32),
    )(emb, W1)


def _sc_gather(table, tokens):
    info = plsc.get_sparse_core_info()
    nw = info.num_cores * info.num_subcores
    bpw = NND // nw
    half = bpw // 2

    mesh = plsc.VectorSubcoreMesh(core_axis_name="c", subcore_axis_name="s")

    @functools.partial(
        pl.kernel,
        mesh=mesh,
        out_type=jax.ShapeDtypeStruct((NND, 128), jnp.float32),
        scratch_types=[
            pltpu.VMEM((bpw,), jnp.int32),
            pltpu.VMEM((half, 128), jnp.float32),
            pltpu.SemaphoreType.DMA,
        ],
    )
    def k(table_hbm, idx_hbm, out_hbm, idx_v, rows_v, sem):
        wid = lax.axis_index("s") * info.num_cores + lax.axis_index("c")
        base = wid * bpw
        pltpu.sync_copy(idx_hbm.at[pl.ds(base, bpw)], idx_v)
        for r in range(2):
            pltpu.async_copy(
                table_hbm.at[idx_v.at[pl.ds(r * half, half)]], rows_v, sem
            ).wait()
            pltpu.sync_copy(rows_v, out_hbm.at[pl.ds(base + r * half, half)])

    return k(table, tokens)


def _sc_counts(edges_flat, zeros_blk):
    info = plsc.get_sparse_core_info()
    nw = info.num_cores * info.num_subcores
    gpt = SCG // nw
    rl = 16
    rounds = gpt // rl
    ewords = 2 * EPG * rl

    rrows = rl * S

    mesh = plsc.VectorSubcoreMesh(core_axis_name="c", subcore_axis_name="s")

    @functools.partial(
        pl.kernel,
        mesh=mesh,
        out_type=jax.ShapeDtypeStruct((SCG * S, 128), jnp.float32),
        scratch_types=[
            pltpu.VMEM((ewords,), jnp.int32),
            pltpu.VMEM((rrows, 128), jnp.float32),
        ],
        compiler_params=pltpu.CompilerParams(needs_layout_passes=False),
    )
    def k(edges_hbm, zeros_hbm, out_hbm, ebuf, cbuf):
        wid = lax.axis_index("s") * info.num_cores + lax.axis_index("c")
        lanes = lax.iota(jnp.int32, 16)
        lane_base = lanes * (2 * EPG)
        lane_row = lanes * S
        ones16 = jnp.ones((16,), jnp.float32)
        for r in range(rounds):
            blk = wid * rounds + r
            pltpu.sync_copy(edges_hbm.at[pl.ds(blk * ewords, ewords)], ebuf)
            pltpu.sync_copy(zeros_hbm, cbuf)

            def edge_body(e, _):
                s_v = plsc.load_gather(ebuf, [lane_base + e])
                d_v = plsc.load_gather(ebuf, [lane_base + (EPG + e)])
                plsc.addupdate_scatter(cbuf, [lane_row + s_v, d_v], ones16)
                return ()

            lax.fori_loop(0, EPG, edge_body, ())
            pltpu.sync_copy(cbuf, out_hbm.at[pl.ds(blk * rrows, rrows)])

    return k(edges_flat, zeros_blk)


def _pad_rows(m, rows):
    return jnp.concatenate(
        [m, jnp.zeros((rows - m.shape[0],) + m.shape[1:], m.dtype)], axis=0)


def _word_body(ct_ref, xw_ref, b1_ref, g1_ref, be1_ref, attw_ref,
               attb_ref, ctx_ref, w2_ref, b2_ref,
               wout_ref, sents_ref, line_ref):
    ct_rows = ct_ref[...]
    xw_rows = xw_ref[...]
    ct3 = ct_rows.reshape(GB, S, 128)
    deg_l = jnp.sum(ct3, axis=1, keepdims=True) + 1.0
    dinv_l = lax.rsqrt(deg_l)
    dinv_r3 = jnp.swapaxes(dinv_l, 1, 2)[:, :S, :]

    y3 = xw_rows.reshape(GB, S, 128) * dinv_r3
    cy = lax.dot_general(ct3, y3, (((1,), (1,)), ((0,), (0,))),
                         preferred_element_type=jnp.float32)
    b1p = _pad_rows(b1_ref[...], 128)
    wo = (cy[:, :S, :] + y3) * dinv_r3 + b1p[None, None, :]

    mu = jnp.sum(wo, axis=2, keepdims=True) * (1.0 / H1)
    var = jnp.sum(wo * wo, axis=2, keepdims=True) * (1.0 / H1) - mu * mu
    g1p = _pad_rows(g1_ref[...], 128)
    be1p = _pad_rows(be1_ref[...], 128)
    normed = (wo - mu) * lax.rsqrt(var + 1e-5) * g1p[None, None, :] \
        + be1p[None, None, :]

    attwp = _pad_rows(attw_ref[...], 128)
    t = jnp.tanh(jnp.dot(normed.reshape(GB * S, 128), attwp,
                         preferred_element_type=jnp.float32)
                 + attb_ref[...][None, :])
    t3 = t.reshape(GB, S, A1)
    att = jnp.sum(t3 * ctx_ref[...][None, None, :], axis=2, keepdims=True)
    mx = jnp.max(att, axis=1, keepdims=True)
    ex = jnp.exp(att - mx)
    w = ex / jnp.sum(ex, axis=1, keepdims=True)
    wout_ref[...] = w

    sents = wo * w
    sents_ref[...] = sents[:, :, :H1]

    uc = jnp.sum(ct3 * dinv_l, axis=2, keepdims=True)
    coeff = (uc + dinv_r3) * dinv_r3
    vsum = jnp.sum(sents * coeff, axis=1)
    w2p = _pad_rows(w2_ref[...], 128)
    line = jnp.dot(vsum, w2p, preferred_element_type=jnp.float32) \
        + float(S) * b2_ref[...][None, :]
    line_ref[...] = line


def _sent_body(line_ref, g2_ref, be2_ref, sattw_ref, sattb_ref, sctx_ref,
               fcw_ref, fcb_ref, sw_ref, scores_ref):
    line = line_ref[...]
    mu = jnp.mean(line, axis=1, keepdims=True)
    var = jnp.mean((line - mu) ** 2, axis=1, keepdims=True)
    normed = (line - mu) * lax.rsqrt(var + 1e-5) * g2_ref[...][None, :] \
        + be2_ref[...][None, :]
    t = jnp.tanh(jnp.dot(normed, sattw_ref[...],
                         preferred_element_type=jnp.float32)
                 + sattb_ref[...][None, :])
    att = jnp.sum(t * sctx_ref[...][None, :], axis=1, keepdims=True)
    att3 = att.reshape(NF, NL, 1)
    mx = jnp.max(att3, axis=1, keepdims=True)
    ex = jnp.exp(att3 - mx)
    sw = ex / jnp.sum(ex, axis=1, keepdims=True)
    sw_ref[...] = sw
    code = jnp.sum(line.reshape(NF, NL, H2) * sw, axis=1)
    scores = jnp.dot(code, fcw_ref[...], preferred_element_type=jnp.float32) \
        + fcb_ref[...][None, :]
    scores_ref[...] = 1.0 / (1.0 + jnp.exp(-scores))


def kernel(code_tensor, word_edge, line_edge, emb, W1, b1, g1, be1, attW,
           attb, ctx, W2, b2, g2, be2, sattW, sattb, sctx, fcW, fcb):
    del line_edge
    tokens = code_tensor.reshape(NND).astype(jnp.int32)
    embw = _tc_embw(emb, W1)
    xw = _sc_gather(embw, tokens)

    edges_flat = word_edge.reshape(-1).astype(jnp.int32)
    zeros_blk = jnp.zeros((16 * S, 128), jnp.float32)
    counts = _sc_counts(edges_flat, zeros_blk)

    nsteps = SCG // GB
    wspec = [
        pl.BlockSpec((H1,), lambda i: (0,)),
        pl.BlockSpec((H1,), lambda i: (0,)),
        pl.BlockSpec((H1,), lambda i: (0,)),
        pl.BlockSpec((H1, A1), lambda i: (0, 0)),
        pl.BlockSpec((A1,), lambda i: (0,)),
        pl.BlockSpec((A1,), lambda i: (0,)),
        pl.BlockSpec((H1, H2), lambda i: (0, 0)),
        pl.BlockSpec((H2,), lambda i: (0,)),
    ]
    wout, sents, line_out = pl.pallas_call(
        _word_body,
        grid=(nsteps,),
        in_specs=[
            pl.BlockSpec((GB * S, 128), lambda i: (i, 0)),
            pl.BlockSpec((GB * S, 128), lambda i: (i, 0)),
        ] + wspec,
        out_specs=[
            pl.BlockSpec((GB, S, 1), lambda i: (i, 0, 0)),
            pl.BlockSpec((GB, S, H1), lambda i: (i, 0, 0)),
            pl.BlockSpec((GB, H2), lambda i: (i, 0)),
        ],
        out_shape=[
            jax.ShapeDtypeStruct((SCG, S, 1), jnp.float32),
            jax.ShapeDtypeStruct((SCG, S, H1), jnp.float32),
            jax.ShapeDtypeStruct((SCG, H2), jnp.float32),
        ],
    )(counts, xw, b1, g1, be1, attW, attb, ctx, W2, b2)

    sw, scores = pl.pallas_call(
        _sent_body,
        out_shape=[
            jax.ShapeDtypeStruct((NF, NL, 1), jnp.float32),
            jax.ShapeDtypeStruct((NF, 1), jnp.float32),
        ],
    )(line_out, g2, be2, sattW, sattb, sctx, fcW, fcb)

    word_att = wout.reshape(NF, NL, S)
    return (scores, word_att, sw.reshape(NF, NL), sents)


# speedup vs baseline: 73.4057x; 1.0059x over previous
import functools

import jax
import jax.numpy as jnp
from jax import lax
from jax.experimental import pallas as pl
from jax.experimental.pallas import tpu as pltpu
from jax.experimental.pallas import tpu_sc as plsc

NF = 16
NL = 64
S = 50
SCG = NF * NL
NND = SCG * S
EPG = 800
H1 = 64
H2 = 64
A1 = 32
A2 = 32
GB = 16


def _embw_body(emb_ref, w1_ref, out_ref):
    out_ref[:, :H1] = jnp.dot(emb_ref[...], w1_ref[...],
                              preferred_element_type=jnp.float32)
    out_ref[:, H1:] = jnp.zeros_like(out_ref[:, H1:])


def _tc_embw(emb, W1):
    v = emb.shape[0]
    vb = 2000
    return pl.pallas_call(
        _embw_body,
        grid=(v // vb,),
        in_specs=[
            pl.BlockSpec((vb, S), lambda i: (i, 0)),
            pl.BlockSpec((S, H1), lambda i: (0, 0)),
        ],
        out_specs=pl.BlockSpec((vb, 128), lambda i: (i, 0)),
        out_shape=jax.ShapeDtypeStruct((v, 128), jnp.float32),
    )(emb, W1)


def _sc_gather(table, tokens):
    info = plsc.get_sparse_core_info()
    nw = info.num_cores * info.num_subcores
    bpw = NND // nw
    half = bpw // 2

    mesh = plsc.VectorSubcoreMesh(core_axis_name="c", subcore_axis_name="s")

    @functools.partial(
        pl.kernel,
        mesh=mesh,
        out_type=jax.ShapeDtypeStruct((NND, 128), jnp.float32),
        scratch_types=[
            pltpu.VMEM((bpw,), jnp.int32),
            pltpu.VMEM((half, 128), jnp.float32),
            pltpu.SemaphoreType.DMA,
        ],
    )
    def k(table_hbm, idx_hbm, out_hbm, idx_v, rows_v, sem):
        wid = lax.axis_index("s") * info.num_cores + lax.axis_index("c")
        base = wid * bpw
        pltpu.sync_copy(idx_hbm.at[pl.ds(base, bpw)], idx_v)
        for r in range(2):
            pltpu.async_copy(
                table_hbm.at[idx_v.at[pl.ds(r * half, half)]], rows_v, sem
            ).wait()
            pltpu.sync_copy(rows_v, out_hbm.at[pl.ds(base + r * half, half)])

    return k(table, tokens)


def _sc_counts(edges_flat, zeros_blk):
    info = plsc.get_sparse_core_info()
    nw = info.num_cores * info.num_subcores
    gpt = SCG // nw
    rl = 16
    rounds = gpt // rl
    ewords = 2 * EPG * rl

    rrows = rl * S

    mesh = plsc.VectorSubcoreMesh(core_axis_name="c", subcore_axis_name="s")

    @functools.partial(
        pl.kernel,
        mesh=mesh,
        out_type=jax.ShapeDtypeStruct((SCG * S, 128), jnp.float32),
        scratch_types=[
            pltpu.VMEM((ewords,), jnp.int32),
            pltpu.VMEM((rrows, 128), jnp.float32),
        ],
        compiler_params=pltpu.CompilerParams(needs_layout_passes=False),
    )
    def k(edges_hbm, zeros_hbm, out_hbm, ebuf, cbuf):
        wid = lax.axis_index("s") * info.num_cores + lax.axis_index("c")
        lanes = lax.iota(jnp.int32, 16)
        lane_base = lanes * (2 * EPG)
        lane_row = lanes * S
        ones16 = jnp.ones((16,), jnp.float32)
        for r in range(rounds):
            blk = wid * rounds + r
            pltpu.sync_copy(edges_hbm.at[pl.ds(blk * ewords, ewords)], ebuf)
            pltpu.sync_copy(zeros_hbm, cbuf)

            def edge_body(e, _):
                s_v = plsc.load_gather(ebuf, [lane_base + e])
                d_v = plsc.load_gather(ebuf, [lane_base + (EPG + e)])
                plsc.addupdate_scatter(cbuf, [lane_row + s_v, d_v], ones16)
                return ()

            lax.fori_loop(0, EPG, edge_body, (), unroll=8)
            pltpu.sync_copy(cbuf, out_hbm.at[pl.ds(blk * rrows, rrows)])

    return k(edges_flat, zeros_blk)


def _pad_rows(m, rows):
    return jnp.concatenate(
        [m, jnp.zeros((rows - m.shape[0],) + m.shape[1:], m.dtype)], axis=0)


def _word_body(ct_ref, xw_ref, b1_ref, g1_ref, be1_ref, attw_ref,
               attb_ref, ctx_ref, w2_ref, b2_ref,
               wout_ref, sents_ref, line_ref):
    ct_rows = ct_ref[...]
    xw_rows = xw_ref[...]
    ct3 = ct_rows.reshape(GB, S, 128)
    deg_l = jnp.sum(ct3, axis=1, keepdims=True) + 1.0
    dinv_l = lax.rsqrt(deg_l)
    dinv_r3 = jnp.swapaxes(dinv_l, 1, 2)[:, :S, :]

    y3 = xw_rows.reshape(GB, S, 128) * dinv_r3
    cy = lax.dot_general(ct3, y3, (((1,), (1,)), ((0,), (0,))),
                         preferred_element_type=jnp.float32)
    b1p = _pad_rows(b1_ref[...], 128)
    wo = (cy[:, :S, :] + y3) * dinv_r3 + b1p[None, None, :]

    mu = jnp.sum(wo, axis=2, keepdims=True) * (1.0 / H1)
    var = jnp.sum(wo * wo, axis=2, keepdims=True) * (1.0 / H1) - mu * mu
    g1p = _pad_rows(g1_ref[...], 128)
    be1p = _pad_rows(be1_ref[...], 128)
    normed = (wo - mu) * lax.rsqrt(var + 1e-5) * g1p[None, None, :] \
        + be1p[None, None, :]

    attwp = _pad_rows(attw_ref[...], 128)
    t = jnp.tanh(jnp.dot(normed.reshape(GB * S, 128), attwp,
                         preferred_element_type=jnp.float32)
                 + attb_ref[...][None, :])
    t3 = t.reshape(GB, S, A1)
    att = jnp.sum(t3 * ctx_ref[...][None, None, :], axis=2, keepdims=True)
    mx = jnp.max(att, axis=1, keepdims=True)
    ex = jnp.exp(att - mx)
    w = ex / jnp.sum(ex, axis=1, keepdims=True)
    wout_ref[...] = w

    sents = wo * w
    sents_ref[...] = sents[:, :, :H1]

    uc = jnp.sum(ct3 * dinv_l, axis=2, keepdims=True)
    coeff = (uc + dinv_r3) * dinv_r3
    vsum = jnp.sum(sents * coeff, axis=1)
    w2p = _pad_rows(w2_ref[...], 128)
    line = jnp.dot(vsum, w2p, preferred_element_type=jnp.float32) \
        + float(S) * b2_ref[...][None, :]
    line_ref[...] = line


def _sent_body(line_ref, g2_ref, be2_ref, sattw_ref, sattb_ref, sctx_ref,
               fcw_ref, fcb_ref, sw_ref, scores_ref):
    line = line_ref[...]
    mu = jnp.mean(line, axis=1, keepdims=True)
    var = jnp.mean((line - mu) ** 2, axis=1, keepdims=True)
    normed = (line - mu) * lax.rsqrt(var + 1e-5) * g2_ref[...][None, :] \
        + be2_ref[...][None, :]
    t = jnp.tanh(jnp.dot(normed, sattw_ref[...],
                         preferred_element_type=jnp.float32)
                 + sattb_ref[...][None, :])
    att = jnp.sum(t * sctx_ref[...][None, :], axis=1, keepdims=True)
    att3 = att.reshape(NF, NL, 1)
    mx = jnp.max(att3, axis=1, keepdims=True)
    ex = jnp.exp(att3 - mx)
    sw = ex / jnp.sum(ex, axis=1, keepdims=True)
    sw_ref[...] = sw
    code = jnp.sum(line.reshape(NF, NL, H2) * sw, axis=1)
    scores = jnp.dot(code, fcw_ref[...], preferred_element_type=jnp.float32) \
        + fcb_ref[...][None, :]
    scores_ref[...] = 1.0 / (1.0 + jnp.exp(-scores))


def kernel(code_tensor, word_edge, line_edge, emb, W1, b1, g1, be1, attW,
           attb, ctx, W2, b2, g2, be2, sattW, sattb, sctx, fcW, fcb):
    del line_edge
    tokens = code_tensor.reshape(NND).astype(jnp.int32)
    edges_flat = word_edge.reshape(-1).astype(jnp.int32)
    edges_flat, W1b, emb_b = lax.optimization_barrier((edges_flat, W1, emb))
    zeros_blk = jnp.zeros((16 * S, 128), jnp.float32)
    counts = _sc_counts(edges_flat, zeros_blk)

    embw = _tc_embw(emb_b, W1b)
    xw = _sc_gather(embw, tokens)

    nsteps = SCG // GB
    wspec = [
        pl.BlockSpec((H1,), lambda i: (0,)),
        pl.BlockSpec((H1,), lambda i: (0,)),
        pl.BlockSpec((H1,), lambda i: (0,)),
        pl.BlockSpec((H1, A1), lambda i: (0, 0)),
        pl.BlockSpec((A1,), lambda i: (0,)),
        pl.BlockSpec((A1,), lambda i: (0,)),
        pl.BlockSpec((H1, H2), lambda i: (0, 0)),
        pl.BlockSpec((H2,), lambda i: (0,)),
    ]
    wout, sents, line_out = pl.pallas_call(
        _word_body,
        grid=(nsteps,),
        in_specs=[
            pl.BlockSpec((GB * S, 128), lambda i: (i, 0)),
            pl.BlockSpec((GB * S, 128), lambda i: (i, 0)),
        ] + wspec,
        out_specs=[
            pl.BlockSpec((GB, S, 1), lambda i: (i, 0, 0)),
            pl.BlockSpec((GB, S, H1), lambda i: (i, 0, 0)),
            pl.BlockSpec((GB, H2), lambda i: (i, 0)),
        ],
        out_shape=[
            jax.ShapeDtypeStruct((SCG, S, 1), jnp.float32),
            jax.ShapeDtypeStruct((SCG, S, H1), jnp.float32),
            jax.ShapeDtypeStruct((SCG, H2), jnp.float32),
        ],
    )(counts, xw, b1, g1, be1, attW, attb, ctx, W2, b2)

    sw, scores = pl.pallas_call(
        _sent_body,
        out_shape=[
            jax.ShapeDtypeStruct((NF, NL, 1), jnp.float32),
            jax.ShapeDtypeStruct((NF, 1), jnp.float32),
        ],
    )(line_out, g2, be2, sattW, sattb, sctx, fcW, fcb)

    word_att = wout.reshape(NF, NL, S)
    return (scores, word_att, sw.reshape(NF, NL), sents)


# speedup vs baseline: 85.8090x; 1.1690x over previous
import functools

import jax
import jax.numpy as jnp
from jax import lax
from jax.experimental import pallas as pl
from jax.experimental.pallas import tpu as pltpu
from jax.experimental.pallas import tpu_sc as plsc

NF = 16
NL = 64
S = 50
SCG = NF * NL
NND = SCG * S
EPG = 800
H1 = 64
H2 = 64
A1 = 32
A2 = 32
GB = 16


def _embw_body(emb_ref, w1_ref, out_ref):
    out_ref[:, :H1] = jnp.dot(emb_ref[...], w1_ref[...],
                              preferred_element_type=jnp.float32)
    out_ref[:, H1:] = jnp.zeros_like(out_ref[:, H1:])


def _tc_embw(emb, W1):
    v = emb.shape[0]
    vb = 2000
    return pl.pallas_call(
        _embw_body,
        grid=(v // vb,),
        in_specs=[
            pl.BlockSpec((vb, S), lambda i: (i, 0)),
            pl.BlockSpec((S, H1), lambda i: (0, 0)),
        ],
        out_specs=pl.BlockSpec((vb, 128), lambda i: (i, 0)),
        out_shape=jax.ShapeDtypeStruct((v, 128), jnp.float32),
    )(emb, W1)


def _sc_gather(table, tokens):
    info = plsc.get_sparse_core_info()
    nw = info.num_cores * info.num_subcores
    bpw = NND // nw
    half = bpw // 2

    mesh = plsc.VectorSubcoreMesh(core_axis_name="c", subcore_axis_name="s")

    @functools.partial(
        pl.kernel,
        mesh=mesh,
        out_type=jax.ShapeDtypeStruct((NND, 128), jnp.float32),
        scratch_types=[
            pltpu.VMEM((bpw,), jnp.int32),
            pltpu.VMEM((half, 128), jnp.float32),
            pltpu.SemaphoreType.DMA,
        ],
    )
    def k(table_hbm, idx_hbm, out_hbm, idx_v, rows_v, sem):
        wid = lax.axis_index("s") * info.num_cores + lax.axis_index("c")
        base = wid * bpw
        pltpu.sync_copy(idx_hbm.at[pl.ds(base, bpw)], idx_v)
        for r in range(2):
            pltpu.async_copy(
                table_hbm.at[idx_v.at[pl.ds(r * half, half)]], rows_v, sem
            ).wait()
            pltpu.sync_copy(rows_v, out_hbm.at[pl.ds(base + r * half, half)])

    return k(table, tokens)


def _sc_counts(edges_flat, zeros_blk):
    info = plsc.get_sparse_core_info()
    nw = info.num_cores * info.num_subcores
    gpt = SCG // nw
    rl = 16
    rounds = gpt // rl
    ewords = 2 * EPG * rl

    rrows = rl * S

    mesh = plsc.VectorSubcoreMesh(core_axis_name="c", subcore_axis_name="s")

    @functools.partial(
        pl.kernel,
        mesh=mesh,
        out_type=jax.ShapeDtypeStruct((SCG * S, 128), jnp.float32),
        scratch_types=[
            pltpu.VMEM((ewords,), jnp.int32),
            pltpu.VMEM((rrows, 128), jnp.float32),
        ],
        compiler_params=pltpu.CompilerParams(needs_layout_passes=False),
    )
    def k(edges_hbm, zeros_hbm, out_hbm, ebuf, cbuf):
        wid = lax.axis_index("s") * info.num_cores + lax.axis_index("c")
        lanes = lax.iota(jnp.int32, 16)
        lane_base = lanes * (2 * EPG)
        lane_row = lanes * S
        ones16 = jnp.ones((16,), jnp.float32)
        for r in range(rounds):
            blk = wid * rounds + r
            pltpu.sync_copy(edges_hbm.at[pl.ds(blk * ewords, ewords)], ebuf)
            pltpu.sync_copy(zeros_hbm, cbuf)

            @plsc.parallel_loop(0, EPG, 1, unroll=8)
            def edge_body(e):
                s_v = plsc.load_gather(ebuf, [lane_base + e])
                d_v = plsc.load_gather(ebuf, [lane_base + (EPG + e)])
                plsc.addupdate_scatter(cbuf, [lane_row + s_v, d_v], ones16)
            pltpu.sync_copy(cbuf, out_hbm.at[pl.ds(blk * rrows, rrows)])

    return k(edges_flat, zeros_blk)


def _pad_rows(m, rows):
    return jnp.concatenate(
        [m, jnp.zeros((rows - m.shape[0],) + m.shape[1:], m.dtype)], axis=0)


def _word_body(ct_ref, xw_ref, b1_ref, g1_ref, be1_ref, attw_ref,
               attb_ref, ctx_ref, w2_ref, b2_ref,
               wout_ref, sents_ref, line_ref):
    ct_rows = ct_ref[...]
    xw_rows = xw_ref[...]
    ct3 = ct_rows.reshape(GB, S, 128)
    deg_l = jnp.sum(ct3, axis=1, keepdims=True) + 1.0
    dinv_l = lax.rsqrt(deg_l)
    dinv_r3 = jnp.swapaxes(dinv_l, 1, 2)[:, :S, :]

    y3 = xw_rows.reshape(GB, S, 128) * dinv_r3
    cy = lax.dot_general(ct3, y3, (((1,), (1,)), ((0,), (0,))),
                         preferred_element_type=jnp.float32)
    b1p = _pad_rows(b1_ref[...], 128)
    wo = (cy[:, :S, :] + y3) * dinv_r3 + b1p[None, None, :]

    mu = jnp.sum(wo, axis=2, keepdims=True) * (1.0 / H1)
    var = jnp.sum(wo * wo, axis=2, keepdims=True) * (1.0 / H1) - mu * mu
    g1p = _pad_rows(g1_ref[...], 128)
    be1p = _pad_rows(be1_ref[...], 128)
    normed = (wo - mu) * lax.rsqrt(var + 1e-5) * g1p[None, None, :] \
        + be1p[None, None, :]

    attwp = _pad_rows(attw_ref[...], 128)
    t = jnp.tanh(jnp.dot(normed.reshape(GB * S, 128), attwp,
                         preferred_element_type=jnp.float32)
                 + attb_ref[...][None, :])
    t3 = t.reshape(GB, S, A1)
    att = jnp.sum(t3 * ctx_ref[...][None, None, :], axis=2, keepdims=True)
    mx = jnp.max(att, axis=1, keepdims=True)
    ex = jnp.exp(att - mx)
    w = ex / jnp.sum(ex, axis=1, keepdims=True)
    wout_ref[...] = w

    sents = wo * w
    sents_ref[...] = sents[:, :, :H1]

    uc = jnp.sum(ct3 * dinv_l, axis=2, keepdims=True)
    coeff = (uc + dinv_r3) * dinv_r3
    vsum = jnp.sum(sents * coeff, axis=1)
    w2p = _pad_rows(w2_ref[...], 128)
    line = jnp.dot(vsum, w2p, preferred_element_type=jnp.float32) \
        + float(S) * b2_ref[...][None, :]
    line_ref[...] = line


def _sent_body(line_ref, g2_ref, be2_ref, sattw_ref, sattb_ref, sctx_ref,
               fcw_ref, fcb_ref, sw_ref, scores_ref):
    line = line_ref[...]
    mu = jnp.mean(line, axis=1, keepdims=True)
    var = jnp.mean((line - mu) ** 2, axis=1, keepdims=True)
    normed = (line - mu) * lax.rsqrt(var + 1e-5) * g2_ref[...][None, :] \
        + be2_ref[...][None, :]
    t = jnp.tanh(jnp.dot(normed, sattw_ref[...],
                         preferred_element_type=jnp.float32)
                 + sattb_ref[...][None, :])
    att = jnp.sum(t * sctx_ref[...][None, :], axis=1, keepdims=True)
    att3 = att.reshape(NF, NL, 1)
    mx = jnp.max(att3, axis=1, keepdims=True)
    ex = jnp.exp(att3 - mx)
    sw = ex / jnp.sum(ex, axis=1, keepdims=True)
    sw_ref[...] = sw
    code = jnp.sum(line.reshape(NF, NL, H2) * sw, axis=1)
    scores = jnp.dot(code, fcw_ref[...], preferred_element_type=jnp.float32) \
        + fcb_ref[...][None, :]
    scores_ref[...] = 1.0 / (1.0 + jnp.exp(-scores))


def kernel(code_tensor, word_edge, line_edge, emb, W1, b1, g1, be1, attW,
           attb, ctx, W2, b2, g2, be2, sattW, sattb, sctx, fcW, fcb):
    del line_edge
    tokens = code_tensor.reshape(NND).astype(jnp.int32)
    edges_flat = word_edge.reshape(-1).astype(jnp.int32)
    edges_flat, W1b, emb_b = lax.optimization_barrier((edges_flat, W1, emb))
    zeros_blk = jnp.zeros((16 * S, 128), jnp.float32)
    counts = _sc_counts(edges_flat, zeros_blk)

    embw = _tc_embw(emb_b, W1b)
    tokens, counts = lax.optimization_barrier((tokens, counts))
    xw = _sc_gather(embw, tokens)

    nsteps = SCG // GB
    wspec = [
        pl.BlockSpec((H1,), lambda i: (0,)),
        pl.BlockSpec((H1,), lambda i: (0,)),
        pl.BlockSpec((H1,), lambda i: (0,)),
        pl.BlockSpec((H1, A1), lambda i: (0, 0)),
        pl.BlockSpec((A1,), lambda i: (0,)),
        pl.BlockSpec((A1,), lambda i: (0,)),
        pl.BlockSpec((H1, H2), lambda i: (0, 0)),
        pl.BlockSpec((H2,), lambda i: (0,)),
    ]
    wout, sents, line_out = pl.pallas_call(
        _word_body,
        grid=(nsteps,),
        in_specs=[
            pl.BlockSpec((GB * S, 128), lambda i: (i, 0)),
            pl.BlockSpec((GB * S, 128), lambda i: (i, 0)),
        ] + wspec,
        out_specs=[
            pl.BlockSpec((GB, S, 1), lambda i: (i, 0, 0)),
            pl.BlockSpec((GB, S, H1), lambda i: (i, 0, 0)),
            pl.BlockSpec((GB, H2), lambda i: (i, 0)),
        ],
        out_shape=[
            jax.ShapeDtypeStruct((SCG, S, 1), jnp.float32),
            jax.ShapeDtypeStruct((SCG, S, H1), jnp.float32),
            jax.ShapeDtypeStruct((SCG, H2), jnp.float32),
        ],
    )(counts, xw, b1, g1, be1, attW, attb, ctx, W2, b2)

    sw, scores = pl.pallas_call(
        _sent_body,
        out_shape=[
            jax.ShapeDtypeStruct((NF, NL, 1), jnp.float32),
            jax.ShapeDtypeStruct((NF, 1), jnp.float32),
        ],
    )(line_out, g2, be2, sattW, sattb, sctx, fcW, fcb)

    word_att = wout.reshape(NF, NL, S)
    return (scores, word_att, sw.reshape(NF, NL), sents)


# speedup vs baseline: 96.6535x; 1.1264x over previous
import functools

import jax
import jax.numpy as jnp
from jax import lax
from jax.experimental import pallas as pl
from jax.experimental.pallas import tpu as pltpu
from jax.experimental.pallas import tpu_sc as plsc

NF = 16
NL = 64
S = 50
SCG = NF * NL
NND = SCG * S
EPG = 800
H1 = 64
H2 = 64
A1 = 32
A2 = 32
GB = 16


def _embw_body(embt_ref, w1_ref, out_ref):
    out_ref[:, :H1] = lax.dot_general(
        embt_ref[...], w1_ref[...], (((0,), (0,)), ((), ())),
        preferred_element_type=jnp.float32)
    out_ref[:, H1:] = jnp.zeros_like(out_ref[:, H1:])


def _tc_embw(embt, W1):
    v = embt.shape[1]
    vb = 8192
    return pl.pallas_call(
        _embw_body,
        grid=(pl.cdiv(v, vb),),
        in_specs=[
            pl.BlockSpec((S, vb), lambda i: (0, i)),
            pl.BlockSpec((S, H1), lambda i: (0, 0)),
        ],
        out_specs=pl.BlockSpec((vb, 128), lambda i: (i, 0)),
        out_shape=jax.ShapeDtypeStruct((v, 128), jnp.float32),
        compiler_params=pltpu.CompilerParams(
            fuse_transposed_lhs_in_matmul=True),
    )(embt, W1)


def _sc_gather(table, tokens):
    info = plsc.get_sparse_core_info()
    nw = info.num_cores * info.num_subcores
    bpw = NND // nw
    half = bpw // 2

    mesh = plsc.VectorSubcoreMesh(core_axis_name="c", subcore_axis_name="s")

    @functools.partial(
        pl.kernel,
        mesh=mesh,
        out_type=jax.ShapeDtypeStruct((NND, 128), jnp.float32),
        scratch_types=[
            pltpu.VMEM((bpw,), jnp.int32),
            pltpu.VMEM((half, 128), jnp.float32),
            pltpu.SemaphoreType.DMA,
        ],
    )
    def k(table_hbm, idx_hbm, out_hbm, idx_v, rows_v, sem):
        wid = lax.axis_index("s") * info.num_cores + lax.axis_index("c")
        base = wid * bpw
        pltpu.sync_copy(idx_hbm.at[pl.ds(base, bpw)], idx_v)
        for r in range(2):
            pltpu.async_copy(
                table_hbm.at[idx_v.at[pl.ds(r * half, half)]], rows_v, sem
            ).wait()
            pltpu.sync_copy(rows_v, out_hbm.at[pl.ds(base + r * half, half)])

    return k(table, tokens)


def _sc_counts(edges_flat, zeros_blk):
    info = plsc.get_sparse_core_info()
    nw = info.num_cores * info.num_subcores
    gpt = SCG // nw
    rl = 16
    rounds = gpt // rl
    ewords = 2 * EPG * rl

    rrows = rl * S

    mesh = plsc.VectorSubcoreMesh(core_axis_name="c", subcore_axis_name="s")

    @functools.partial(
        pl.kernel,
        mesh=mesh,
        out_type=jax.ShapeDtypeStruct((SCG * S, 128), jnp.float32),
        scratch_types=[
            pltpu.VMEM((ewords,), jnp.int32),
            pltpu.VMEM((rrows, 128), jnp.float32),
        ],
        compiler_params=pltpu.CompilerParams(needs_layout_passes=False),
    )
    def k(edges_hbm, zeros_hbm, out_hbm, ebuf, cbuf):
        wid = lax.axis_index("s") * info.num_cores + lax.axis_index("c")
        lanes = lax.iota(jnp.int32, 16)
        lane_base = lanes * (2 * EPG)
        lane_row = lanes * S
        ones16 = jnp.ones((16,), jnp.float32)
        for r in range(rounds):
            blk = wid * rounds + r
            pltpu.sync_copy(edges_hbm.at[pl.ds(blk * ewords, ewords)], ebuf)
            pltpu.sync_copy(zeros_hbm, cbuf)

            @plsc.parallel_loop(0, EPG, 1, unroll=8)
            def edge_body(e):
                s_v = plsc.load_gather(ebuf, [lane_base + e])
                d_v = plsc.load_gather(ebuf, [lane_base + (EPG + e)])
                plsc.addupdate_scatter(cbuf, [lane_row + s_v, d_v], ones16)
            pltpu.sync_copy(cbuf, out_hbm.at[pl.ds(blk * rrows, rrows)])

    return k(edges_flat, zeros_blk)


def _pad_rows(m, rows):
    return jnp.concatenate(
        [m, jnp.zeros((rows - m.shape[0],) + m.shape[1:], m.dtype)], axis=0)


def _word_body(ct_ref, xw_ref, b1_ref, g1_ref, be1_ref, attw_ref,
               attb_ref, ctx_ref, w2_ref, b2_ref,
               wout_ref, sents_ref, line_ref):
    ct_rows = ct_ref[...]
    xw_rows = xw_ref[...]
    ct3 = ct_rows.reshape(GB, S, 128)
    deg_l = jnp.sum(ct3, axis=1, keepdims=True) + 1.0
    dinv_l = lax.rsqrt(deg_l)
    dinv_r3 = jnp.swapaxes(dinv_l, 1, 2)[:, :S, :]

    y3 = xw_rows.reshape(GB, S, 128) * dinv_r3
    cy = lax.dot_general(ct3, y3, (((1,), (1,)), ((0,), (0,))),
                         preferred_element_type=jnp.float32)
    b1p = _pad_rows(b1_ref[...], 128)
    wo = (cy[:, :S, :] + y3) * dinv_r3 + b1p[None, None, :]

    mu = jnp.sum(wo, axis=2, keepdims=True) * (1.0 / H1)
    var = jnp.sum(wo * wo, axis=2, keepdims=True) * (1.0 / H1) - mu * mu
    g1p = _pad_rows(g1_ref[...], 128)
    be1p = _pad_rows(be1_ref[...], 128)
    normed = (wo - mu) * lax.rsqrt(var + 1e-5) * g1p[None, None, :] \
        + be1p[None, None, :]

    attwp = _pad_rows(attw_ref[...], 128)
    t = jnp.tanh(jnp.dot(normed.reshape(GB * S, 128), attwp,
                         preferred_element_type=jnp.float32)
                 + attb_ref[...][None, :])
    t3 = t.reshape(GB, S, A1)
    att = jnp.sum(t3 * ctx_ref[...][None, None, :], axis=2, keepdims=True)
    mx = jnp.max(att, axis=1, keepdims=True)
    ex = jnp.exp(att - mx)
    w = ex / jnp.sum(ex, axis=1, keepdims=True)
    wout_ref[...] = w

    sents = wo * w
    sents_ref[...] = sents[:, :, :H1]

    uc = jnp.sum(ct3 * dinv_l, axis=2, keepdims=True)
    coeff = (uc + dinv_r3) * dinv_r3
    vsum = jnp.sum(sents * coeff, axis=1)
    w2p = _pad_rows(w2_ref[...], 128)
    line = jnp.dot(vsum, w2p, preferred_element_type=jnp.float32) \
        + float(S) * b2_ref[...][None, :]
    line_ref[...] = line


def _sent_body(line_ref, g2_ref, be2_ref, sattw_ref, sattb_ref, sctx_ref,
               fcw_ref, fcb_ref, sw_ref, scores_ref):
    line = line_ref[...]
    mu = jnp.mean(line, axis=1, keepdims=True)
    var = jnp.mean((line - mu) ** 2, axis=1, keepdims=True)
    normed = (line - mu) * lax.rsqrt(var + 1e-5) * g2_ref[...][None, :] \
        + be2_ref[...][None, :]
    t = jnp.tanh(jnp.dot(normed, sattw_ref[...],
                         preferred_element_type=jnp.float32)
                 + sattb_ref[...][None, :])
    att = jnp.sum(t * sctx_ref[...][None, :], axis=1, keepdims=True)
    att3 = att.reshape(NF, NL, 1)
    mx = jnp.max(att3, axis=1, keepdims=True)
    ex = jnp.exp(att3 - mx)
    sw = ex / jnp.sum(ex, axis=1, keepdims=True)
    sw_ref[...] = sw
    code = jnp.sum(line.reshape(NF, NL, H2) * sw, axis=1)
    scores = jnp.dot(code, fcw_ref[...], preferred_element_type=jnp.float32) \
        + fcb_ref[...][None, :]
    scores_ref[...] = 1.0 / (1.0 + jnp.exp(-scores))


def kernel(code_tensor, word_edge, line_edge, emb, W1, b1, g1, be1, attW,
           attb, ctx, W2, b2, g2, be2, sattW, sattb, sctx, fcW, fcb):
    del line_edge
    tokens = code_tensor.reshape(NND).astype(jnp.int32)
    edges_flat = word_edge.reshape(-1).astype(jnp.int32)
    edges_flat, W1b, emb_b = lax.optimization_barrier((edges_flat, W1, emb))
    zeros_blk = jnp.zeros((16 * S, 128), jnp.float32)
    counts = _sc_counts(edges_flat, zeros_blk)

    embw = _tc_embw(emb_b.T, W1b)
    tokens, counts = lax.optimization_barrier((tokens, counts))
    xw = _sc_gather(embw, tokens)

    nsteps = SCG // GB
    wspec = [
        pl.BlockSpec((H1,), lambda i: (0,)),
        pl.BlockSpec((H1,), lambda i: (0,)),
        pl.BlockSpec((H1,), lambda i: (0,)),
        pl.BlockSpec((H1, A1), lambda i: (0, 0)),
        pl.BlockSpec((A1,), lambda i: (0,)),
        pl.BlockSpec((A1,), lambda i: (0,)),
        pl.BlockSpec((H1, H2), lambda i: (0, 0)),
        pl.BlockSpec((H2,), lambda i: (0,)),
    ]
    wout, sents, line_out = pl.pallas_call(
        _word_body,
        grid=(nsteps,),
        in_specs=[
            pl.BlockSpec((GB * S, 128), lambda i: (i, 0)),
            pl.BlockSpec((GB * S, 128), lambda i: (i, 0)),
        ] + wspec,
        out_specs=[
            pl.BlockSpec((GB, S, 1), lambda i: (i, 0, 0)),
            pl.BlockSpec((GB, S, H1), lambda i: (i, 0, 0)),
            pl.BlockSpec((GB, H2), lambda i: (i, 0)),
        ],
        out_shape=[
            jax.ShapeDtypeStruct((SCG, S, 1), jnp.float32),
            jax.ShapeDtypeStruct((SCG, S, H1), jnp.float32),
            jax.ShapeDtypeStruct((SCG, H2), jnp.float32),
        ],
    )(counts, xw, b1, g1, be1, attW, attb, ctx, W2, b2)

    sw, scores = pl.pallas_call(
        _sent_body,
        out_shape=[
            jax.ShapeDtypeStruct((NF, NL, 1), jnp.float32),
            jax.ShapeDtypeStruct((NF, 1), jnp.float32),
        ],
    )(line_out, g2, be2, sattW, sattb, sctx, fcW, fcb)

    word_att = wout.reshape(NF, NL, S)
    return (scores, word_att, sw.reshape(NF, NL), sents)
